# baseline (speedup 1.0000x reference)
import functools

import jax
import jax.numpy as jnp
from jax import lax
from jax.experimental import pallas as pl
from jax.experimental.pallas import tpu as pltpu
from jax.experimental.pallas import tpu_sc as plsc

L = 16
NS = 16
NUM_EXPERTS = 64


def _make_hist_kernel(n_flat: int, num_experts: int):
  E = num_experts
  U = 4
  chunk = n_flat // NS
  assert chunk * NS == n_flat and chunk % (U * L) == 0 and E % L == 0

  mesh = plsc.VectorSubcoreMesh(
      core_axis_name="c", subcore_axis_name="s", num_cores=1, num_subcores=NS)

  @functools.partial(
      pl.kernel,
      out_type=jax.ShapeDtypeStruct((E,), jnp.int32),
      mesh=mesh,
      compiler_params=pltpu.CompilerParams(
          needs_layout_passes=False, use_tc_tiling_on_sc=False,
          skip_device_barrier=True),
      scratch_types=[
          pltpu.VMEM((chunk,), jnp.int32),
          pltpu.VMEM((U * E * L,), jnp.int32),
          pltpu.VMEM((E,), jnp.int32),
          pltpu.VMEM((NS * E,), jnp.int32),
          pltpu.VMEM_SHARED((NS * E,), jnp.int32),
          pltpu.SemaphoreType.DMA,
      ],
  )
  def hist_kernel(ids_hbm, out_hbm, ids_v, hist_v, cnt_v, gbuf_v, shared,
                  sem):
    sid = lax.axis_index("s")
    pltpu.sync_copy(ids_hbm.at[pl.ds(sid * chunk, chunk)], ids_v)

    lanes = lax.iota(jnp.int32, L)
    zeros = jnp.zeros((L,), jnp.int32)
    ones = jnp.ones((L,), jnp.int32)
    for r in range(U * E):
      hist_v[pl.ds(r * L, L)] = zeros

    def body(i, carry):
      for u in range(U):
        v = ids_v[pl.ds(i * (U * L) + u * L, L)]
        plsc.addupdate_scatter(hist_v, [v * L + (u * E * L) + lanes], ones)
      return carry

    lax.fori_loop(0, chunk // (U * L), body, 0)

    for k in range(E // L):
      acc = zeros
      for j in range(L):
        b = (k * L + j) * L
        row = hist_v[pl.ds(b, L)]
        for u in range(1, U):
          row = row + hist_v[pl.ds(u * E * L + b, L)]
        s = jnp.sum(row)
        acc = jnp.where(lanes == j, s, acc)
      cnt_v[pl.ds(k * L, L)] = acc

    pltpu.sync_copy(cnt_v, shared.at[pl.ds(sid * E, E)])
    plsc.subcore_barrier()

    @pl.when(sid == 0)
    def _():
      pltpu.sync_copy(shared, gbuf_v)
      for k in range(E // L):
        acc = zeros
        for s_ in range(NS):
          acc = acc + gbuf_v[pl.ds(s_ * E + k * L, L)]
        cnt_v[pl.ds(k * L, L)] = acc
      pltpu.sync_copy(cnt_v, out_hbm)

  return hist_kernel


def kernel(topk_ids, num_local_experts):
  del num_local_experts
  ids = topk_ids.reshape(-1).astype(jnp.int32)
  hist = _make_hist_kernel(ids.shape[0], NUM_EXPERTS)
  return hist(ids)


# speedup vs baseline: 1.0186x; 1.0186x over previous
import functools

import jax
import jax.numpy as jnp
from jax import lax
from jax.experimental import pallas as pl
from jax.experimental.pallas import tpu as pltpu
from jax.experimental.pallas import tpu_sc as plsc

L = 16
NS = 16
NUM_EXPERTS = 64


def _make_hist_kernel(n_flat: int, num_experts: int):
  E = num_experts
  chunk = n_flat // NS
  assert chunk * NS == n_flat and chunk % L == 0 and E % L == 0

  mesh = plsc.VectorSubcoreMesh(
      core_axis_name="c", subcore_axis_name="s", num_cores=1, num_subcores=NS)

  @functools.partial(
      pl.kernel,
      out_type=jax.ShapeDtypeStruct((E,), jnp.int32),
      mesh=mesh,
      compiler_params=pltpu.CompilerParams(
          needs_layout_passes=False, use_tc_tiling_on_sc=False,
          skip_device_barrier=True),
      scratch_types=[
          pltpu.VMEM((chunk,), jnp.int32),
          pltpu.VMEM((E * L,), jnp.int32),
          pltpu.VMEM((E,), jnp.int32),
          pltpu.VMEM((NS * E,), jnp.int32),
          pltpu.VMEM_SHARED((NS * E,), jnp.int32),
          pltpu.SemaphoreType.DMA,
      ],
  )
  def hist_kernel(ids_hbm, out_hbm, ids_v, hist_v, cnt_v, gbuf_v, shared,
                  sem):
    sid = lax.axis_index("s")
    with jax.named_scope("stage"):
      pltpu.sync_copy(ids_hbm.at[pl.ds(sid * chunk, chunk)], ids_v)

    lanes = lax.iota(jnp.int32, L)
    zeros = jnp.zeros((L,), jnp.int32)
    ones = jnp.ones((L,), jnp.int32)
    with jax.named_scope("zero"):
      for r in range(E):
        hist_v[pl.ds(r * L, L)] = zeros

    def body(i, carry):
      v = ids_v[pl.ds(i * L, L)]
      plsc.addupdate_scatter(hist_v, [v * L + lanes], ones)
      return carry

    with jax.named_scope("scatter"):
      lax.fori_loop(0, chunk // L, body, 0)

    with jax.named_scope("reduce"):
      for k in range(E // L):
        acc = zeros
        for j in range(L):
          s = jnp.sum(hist_v[pl.ds((k * L + j) * L, L)])
          acc = jnp.where(lanes == j, s, acc)
        cnt_v[pl.ds(k * L, L)] = acc

    with jax.named_scope("combine"):
      pltpu.sync_copy(cnt_v, shared.at[pl.ds(sid * E, E)])
      plsc.subcore_barrier()

      @pl.when(sid == 0)
      def _():
        pltpu.sync_copy(shared, gbuf_v)
        for k in range(E // L):
          acc = zeros
          for s_ in range(NS):
            acc = acc + gbuf_v[pl.ds(s_ * E + k * L, L)]
          cnt_v[pl.ds(k * L, L)] = acc
        pltpu.sync_copy(cnt_v, out_hbm)

  return hist_kernel


def kernel(topk_ids, num_local_experts):
  del num_local_experts
  ids = topk_ids.reshape(-1).astype(jnp.int32)
  hist = _make_hist_kernel(ids.shape[0], NUM_EXPERTS)
  return hist(ids)


# speedup vs baseline: 1.0416x; 1.0226x over previous
import functools

import jax
import jax.numpy as jnp
from jax import lax
from jax.experimental import pallas as pl
from jax.experimental.pallas import tpu as pltpu
from jax.experimental.pallas import tpu_sc as plsc

L = 16
NS = 16
W = 128
NUM_EXPERTS = 64


def _make_hist_kernel(n_rows: int, num_experts: int):
  E = num_experts
  rows = n_rows // NS
  assert rows * NS == n_rows and E % L == 0

  mesh = plsc.VectorSubcoreMesh(
      core_axis_name="c", subcore_axis_name="s", num_cores=1, num_subcores=NS)

  @functools.partial(
      pl.kernel,
      out_type=jax.ShapeDtypeStruct((E,), jnp.int32),
      mesh=mesh,
      compiler_params=pltpu.CompilerParams(
          needs_layout_passes=False, use_tc_tiling_on_sc=False,
          skip_device_barrier=True),
      scratch_types=[
          pltpu.VMEM((rows, W), jnp.int32),
          pltpu.VMEM((E * L,), jnp.int32),
          pltpu.VMEM((E,), jnp.int32),
          pltpu.VMEM((NS * E,), jnp.int32),
          pltpu.VMEM_SHARED((NS * E,), jnp.int32),
      ],
  )
  def hist_kernel(ids_hbm, out_hbm, ids_v, hist_v, cnt_v, gbuf_v, shared):
    sid = lax.axis_index("s")
    pltpu.sync_copy(ids_hbm.at[pl.ds(sid * rows, rows)], ids_v)

    lanes = lax.iota(jnp.int32, L)
    zeros = jnp.zeros((L,), jnp.int32)
    ones = jnp.ones((L,), jnp.int32)
    for r in range(E):
      hist_v[pl.ds(r * L, L)] = zeros

    def body(r, carry):
      for c in range(W // L):
        v = ids_v[r, pl.ds(c * L, L)]
        plsc.addupdate_scatter(hist_v, [v * L + lanes], ones)
      return carry

    lax.fori_loop(0, rows, body, 0)

    for k in range(E // L):
      acc = zeros
      for j in range(L):
        s = jnp.sum(hist_v[pl.ds((k * L + j) * L, L)])
        acc = jnp.where(lanes == j, s, acc)
      cnt_v[pl.ds(k * L, L)] = acc

    pltpu.sync_copy(cnt_v, shared.at[pl.ds(sid * E, E)])
    plsc.subcore_barrier()

    @pl.when(sid == 0)
    def _():
      pltpu.sync_copy(shared, gbuf_v)
      for k in range(E // L):
        acc = zeros
        for s_ in range(NS):
          acc = acc + gbuf_v[pl.ds(s_ * E + k * L, L)]
        cnt_v[pl.ds(k * L, L)] = acc
      pltpu.sync_copy(cnt_v, out_hbm)

  return hist_kernel


def kernel(topk_ids, num_local_experts):
  del num_local_experts
  n = topk_ids.shape[0] * topk_ids.shape[1]
  ids = topk_ids.reshape(n // W, W)
  hist = _make_hist_kernel(n // W, NUM_EXPERTS)
  return hist(ids)
